# Initial kernel scaffold; baseline (speedup 1.0000x reference)
import jax
import jax.numpy as jnp
from jax.experimental import pallas as pl


def kernel(input, targets):
    raise NotImplementedError("write your pallas kernel here")


# speedup vs baseline: 1.2794x; 1.2794x over previous
import functools

import jax
import jax.numpy as jnp
from jax import lax
from jax.experimental import pallas as pl
from jax.experimental.pallas import tpu as pltpu
from jax.experimental.pallas import tpu_sc as plsc

B = 128
T = 200
V = 1000
L = 16
CH = 40
NCH = T // CH
NW = 32
NB = B // NW
CHW = CH * V


def _sc_scores(x_flat, tg_flat):
    mesh = plsc.VectorSubcoreMesh(core_axis_name="c", subcore_axis_name="s")

    @functools.partial(
        pl.kernel,
        out_type=jax.ShapeDtypeStruct((B * B,), jnp.float32),
        mesh=mesh,
        compiler_params=pltpu.CompilerParams(needs_layout_passes=False),
        scratch_types=[
            pltpu.VMEM((B * T,), jnp.int32),
            pltpu.VMEM((2 * CHW,), jnp.float32),
            pltpu.VMEM((NB * B,), jnp.float32),
            pltpu.SemaphoreType.DMA,
            pltpu.SemaphoreType.DMA,
        ],
    )
    def k(x_hbm, tg_hbm, out_hbm, tg_v, xbuf, acc, sem0, sem1):
        sems = (sem0, sem1)
        wid = lax.axis_index("s") * 2 + lax.axis_index("c")
        b0 = wid * NB

        pltpu.sync_copy(tg_hbm, tg_v)

        zeros = jnp.zeros((L,), jnp.float32)
        for j in range(NB * B // L):
            acc[pl.ds(j * L, L)] = zeros

        lane = lax.broadcasted_iota(jnp.int32, (L,), 0)
        tg_base = [(lane + j * L) * T for j in range(B // L)]

        def start(kk):
            bb, cc = divmod(kk, NCH)
            src_off = (b0 + bb) * T * V + cc * CHW
            return pltpu.async_copy(
                x_hbm.at[pl.ds(src_off, CHW)],
                xbuf.at[pl.ds((kk % 2) * CHW, CHW)],
                sems[kk % 2],
            )

        total = NB * NCH
        handles = [start(0), None]
        for kk in range(total):
            if kk + 1 < total:
                handles[(kk + 1) % 2] = start(kk + 1)
            handles[kk % 2].wait()
            bb, cc = divmod(kk, NCH)
            bufbase = (kk % 2) * CHW

            def body(tl, carry, bb=bb, cc=cc, bufbase=bufbase):
                tvec = jnp.full((L,), cc * CH + tl, jnp.int32)
                vbase = jnp.full((L,), bufbase + tl * V, jnp.int32)
                for j in range(B // L):
                    tgt = plsc.load_gather(tg_v, [tg_base[j] + tvec])
                    vals = plsc.load_gather(xbuf, [vbase + tgt])
                    vals = jnp.minimum(jnp.maximum(vals, -30.0), 30.0)
                    plsc.addupdate(acc.at[pl.ds(bb * B + j * L, L)], vals)
                return carry

            lax.fori_loop(0, CH, body, 0)

        pltpu.sync_copy(acc, out_hbm.at[pl.ds(b0 * B, NB * B)])

    return k(x_flat, tg_flat)


def _tc_loss(scoresT):
    def body(s_ref, o_ref):
        s = s_ref[...]
        m = jnp.max(s, axis=0, keepdims=True)
        lse = m + jnp.log(jnp.sum(jnp.exp(s - m), axis=0, keepdims=True))
        bi = lax.broadcasted_iota(jnp.int32, (B, B), 0)
        ii = lax.broadcasted_iota(jnp.int32, (B, B), 1)
        num = jnp.sum(jnp.where(bi == ii, s, 0.0), axis=0, keepdims=True)
        o_ref[...] = jnp.sum(num - lse).reshape(1, 1)

    out = pl.pallas_call(
        body, out_shape=jax.ShapeDtypeStruct((1, 1), jnp.float32)
    )(scoresT)
    return out[0, 0]


def kernel(input, targets):
    tg_flat = targets.astype(jnp.int32).reshape(-1)
    x_flat = input.reshape(-1)
    scoresT = _sc_scores(x_flat, tg_flat).reshape(B, B)
    return _tc_loss(scoresT)


# speedup vs baseline: 1.4113x; 1.1031x over previous
import functools

import jax
import jax.numpy as jnp
from jax import lax
from jax.experimental import pallas as pl
from jax.experimental.pallas import tpu as pltpu
from jax.experimental.pallas import tpu_sc as plsc

B = 128
T = 200
V = 1000
L = 16
NW = 32
NB = B // NW
NSLOT = 4
NBL = 8
Q = T // 8
SLOTW = NBL * 8 * L * 8


def _sc_scores(x, xtail, tgT_flat):
    mesh = plsc.VectorSubcoreMesh(core_axis_name="c", subcore_axis_name="s")

    @functools.partial(
        pl.kernel,
        out_type=jax.ShapeDtypeStruct((B * B,), jnp.float32),
        mesh=mesh,
        compiler_params=pltpu.CompilerParams(needs_layout_passes=False),
        scratch_types=[
            pltpu.VMEM((T * B,), jnp.int32),
            pltpu.VMEM((NSLOT * 64, 128), jnp.float32),
            pltpu.VMEM((NB * B,), jnp.float32),
        ]
        + [pltpu.SemaphoreType.DMA] * NSLOT,
    )
    def k(x_hbm, xtail_hbm, tg_hbm, out_hbm, tg_v, xbuf, acc, *sems):
        wid = lax.axis_index("s") * 2 + lax.axis_index("c")
        b0 = wid * NB

        pltpu.sync_copy(tg_hbm, tg_v)

        zeros = jnp.zeros((L,), jnp.float32)
        for j in range(NB * B // L):
            acc[pl.ds(j * L, L)] = zeros

        def issue(b, q, u):
            for bl in range(NBL - 1):
                pltpu.async_copy(
                    x_hbm.at[b, pl.ds(q * 8, 8), pl.ds(bl * 128, 128)],
                    xbuf.at[pl.ds(u * 64 + bl * 8, 8), :],
                    sems[u],
                )
            pltpu.async_copy(
                xtail_hbm.at[b, pl.ds(q * 8, 8), :],
                xbuf.at[pl.ds(u * 64 + (NBL - 1) * 8, 8), :],
                sems[u],
            )

        def drain(u):
            pltpu.make_async_copy(
                x_hbm.at[0, pl.ds(0, 64), pl.ds(0, 128)],
                xbuf.at[pl.ds(u * 64, 64), :],
                sems[u],
            ).wait()

        for u in range(NSLOT):
            issue(b0 + u // Q, u % Q, u)

        total = NB * Q
        ng = -(-total // NSLOT)

        def outer(gg, carry):
            for u in range(NSLOT):
                g = gg * NSLOT + u

                @pl.when(g < total)
                def _(g=g, u=u):
                    drain(u)
                    bb = g // Q
                    q = g - bb * Q
                    for rr in range(8):
                        t = q * 8 + rr
                        base_row = u * 64 + rr
                        for j in range(B // L):
                            tgt = tg_v[pl.ds(t * B + j * L, L)]
                            bl = tgt >> 7
                            row = bl * 8 + base_row
                            col = (tgt & 127) + jnp.where(bl == 7, 24, 0)
                            vals = plsc.load_gather(xbuf, [row, col])
                            vals = jnp.minimum(jnp.maximum(vals, -30.0), 30.0)
                            plsc.addupdate(
                                acc.at[pl.ds(bb * B + j * L, L)], vals)
                    gn = g + NSLOT

                    @pl.when(gn < total)
                    def _():
                        bn = gn // Q
                        issue(b0 + bn, gn - bn * Q, u)

            return carry

        lax.fori_loop(0, ng, outer, 0)

        pltpu.sync_copy(acc, out_hbm.at[pl.ds(b0 * B, NB * B)])

    return k(x, xtail, tgT_flat)


def _tc_loss(scoresT):
    def body(s_ref, o_ref):
        s = s_ref[...]
        m = jnp.max(s, axis=0, keepdims=True)
        lse = m + jnp.log(jnp.sum(jnp.exp(s - m), axis=0, keepdims=True))
        bi = lax.broadcasted_iota(jnp.int32, (B, B), 0)
        ii = lax.broadcasted_iota(jnp.int32, (B, B), 1)
        num = jnp.sum(jnp.where(bi == ii, s, 0.0), axis=0, keepdims=True)
        o_ref[...] = jnp.sum(num - lse).reshape(1, 1)

    out = pl.pallas_call(
        body, out_shape=jax.ShapeDtypeStruct((1, 1), jnp.float32)
    )(scoresT)
    return out[0, 0]


def kernel(input, targets):
    tgT_flat = targets.astype(jnp.int32).T.reshape(-1)
    xtail = lax.slice(input, (0, 0, V - 128), (B, T, V))
    scoresT = _sc_scores(input, xtail, tgT_flat).reshape(B, B)
    return _tc_loss(scoresT)
